# baseline (speedup 1.0000x reference)
import jax
import jax.numpy as jnp
from jax import lax
from jax.experimental import pallas as pl
from jax.experimental.pallas import tpu as pltpu
from jax.experimental.pallas import tpu_sc as plsc

NUM_EXPERTS = 64
TOP_K = 8
BT = 256
NW = 32
L = 16


def _logits_block(x_ref, w_ref, out_ref):
    x = x_ref[...]
    w = w_ref[...]
    lt = lax.dot_general(
        w, x, (((1,), (1,)), ((), ())), preferred_element_type=jnp.float32
    )
    out_ref[...] = lt.reshape(1, 1, NUM_EXPERTS * BT)


def _sc_body(lin_hbm, wout_hbm, iout_hbm, lvm, wvm, ivm):
    nblk = lin_hbm.shape[0]
    blocks_per_w = nblk // NW
    cid = lax.axis_index("c")
    sid = lax.axis_index("s")
    wid = sid * 2 + cid
    lane = lax.broadcasted_iota(jnp.int32, (L,), 0)
    neg_inf = jnp.full((L,), -jnp.inf, dtype=jnp.float32)
    zeros = jnp.zeros((L,), jnp.float32)

    def do_block(c, _):
        blk = wid * blocks_per_w + c
        pltpu.sync_copy(lin_hbm.at[blk, 0], lvm)

        def zrow(r, _):
            for u in range(8):
                wvm[pl.ds((r * 8 + u) * L, L)] = zeros
            return _
        lax.fori_loop(0, BT * NUM_EXPERTS // (8 * L), zrow, 0)

        def group(g, _):
            base = g * L
            toks = base + lane
            vals = []
            idxs = []
            for j in range(TOP_K):
                m = neg_inf
                idx = jnp.zeros((L,), jnp.int32)
                for e in range(NUM_EXPERTS):
                    v = lvm[pl.ds(e * BT + base, L)]
                    upd = v > m
                    m = jnp.where(upd, v, m)
                    idx = jnp.where(upd, jnp.int32(e), idx)
                vals.append(m)
                idxs.append(idx)
                if j + 1 < TOP_K:
                    plsc.store_scatter(lvm, [idx * BT + toks], neg_inf)
            ws = [jnp.exp(v - vals[0]) for v in vals]
            s = ws[0]
            for j in range(1, TOP_K):
                s = s + ws[j]
            for j in range(TOP_K):
                plsc.store_scatter(
                    wvm, [toks * NUM_EXPERTS + idxs[j]], ws[j] / s
                )
                plsc.store_scatter(
                    ivm, [toks * TOP_K + j], idxs[j]
                )
            return _
        lax.fori_loop(0, BT // L, group, 0)

        pltpu.sync_copy(wvm, wout_hbm.at[pl.ds(blk * BT * NUM_EXPERTS,
                                               BT * NUM_EXPERTS)])
        pltpu.sync_copy(ivm, iout_hbm.at[pl.ds(blk * BT * TOP_K,
                                               BT * TOP_K)])
        return 0

    lax.fori_loop(0, blocks_per_w, do_block, 0)


@jax.jit
def kernel(input, W):
    b, s, d = input.shape
    e = W.shape[0]
    mesh = plsc.VectorSubcoreMesh(core_axis_name="c", subcore_axis_name="s")
    nblk = s // BT

    w_slices = []
    i_slices = []
    for k in range(b):
        logits_blocked = pl.pallas_call(
            _logits_block,
            grid=(nblk,),
            in_specs=[
                pl.BlockSpec((BT, d), lambda i: (i, 0)),
                pl.BlockSpec((e, d), lambda i: (0, 0)),
            ],
            out_specs=pl.BlockSpec((1, 1, e * BT), lambda i: (i, 0, 0)),
            out_shape=jax.ShapeDtypeStruct((nblk, 1, e * BT), jnp.float32),
        )(input[k], W)

        wk, ik = pl.kernel(
            _sc_body,
            out_type=[
                jax.ShapeDtypeStruct((s * e,), jnp.float32),
                jax.ShapeDtypeStruct((s * TOP_K,), jnp.int32),
            ],
            mesh=mesh,
            compiler_params=pltpu.CompilerParams(needs_layout_passes=False),
            scratch_types=[
                pltpu.VMEM((e * BT,), jnp.float32),
                pltpu.VMEM((BT * e,), jnp.float32),
                pltpu.VMEM((BT * TOP_K,), jnp.int32),
            ],
        )(logits_blocked)
        w_slices.append(wk.reshape(s, e))
        i_slices.append(ik.reshape(s, TOP_K))

    return jnp.stack(w_slices), jnp.stack(i_slices)


# speedup vs baseline: 2.2884x; 2.2884x over previous
import jax
import jax.numpy as jnp
from jax import lax
from jax.experimental import pallas as pl
from jax.experimental.pallas import tpu as pltpu
from jax.experimental.pallas import tpu_sc as plsc

NUM_EXPERTS = 64
TOP_K = 8
BT = 512
TCB = 1024
NW = 32
L = 16


def _logits_block(x_ref, w_ref, out_ref):
    x = x_ref[...]
    w = w_ref[...]
    lt = lax.dot_general(
        w, x, (((1,), (1,)), ((), ())), preferred_element_type=jnp.float32
    )
    out_ref[0] = lt[:, :BT].reshape(1, NUM_EXPERTS * BT)
    out_ref[1] = lt[:, BT:].reshape(1, NUM_EXPERTS * BT)


def _sc_body(lin_hbm, wout_hbm, iout_hbm, lvm, wvm, ivm):
    nblk = lin_hbm.shape[0]
    blocks_per_w = nblk // NW
    cid = lax.axis_index("c")
    sid = lax.axis_index("s")
    wid = sid * 2 + cid
    lane = lax.broadcasted_iota(jnp.int32, (L,), 0)
    neg_inf = jnp.full((L,), -jnp.inf, dtype=jnp.float32)
    zeros = jnp.zeros((L,), jnp.float32)

    def do_block(c, _):
        blk = wid * blocks_per_w + c
        pltpu.sync_copy(lin_hbm.at[blk, 0], lvm)

        def zrow(r, _):
            for u in range(8):
                wvm[pl.ds((r * 8 + u) * L, L)] = zeros
            return _
        lax.fori_loop(0, BT * NUM_EXPERTS // (8 * L), zrow, 0)

        def group(g, _):
            base = g * L
            toks = base + lane
            vals = []
            idxs = []
            for j in range(TOP_K):
                m = neg_inf
                idx = jnp.zeros((L,), jnp.int32)
                for e in range(NUM_EXPERTS):
                    v = lvm[pl.ds(e * BT + base, L)]
                    upd = v > m
                    m = jnp.where(upd, v, m)
                    idx = jnp.where(upd, jnp.int32(e), idx)
                vals.append(m)
                idxs.append(idx)
                if j + 1 < TOP_K:
                    plsc.store_scatter(lvm, [idx * BT + toks], neg_inf)
            ws = [jnp.exp(v - vals[0]) for v in vals]
            s = ws[0]
            for j in range(1, TOP_K):
                s = s + ws[j]
            for j in range(TOP_K):
                plsc.store_scatter(
                    wvm, [toks * NUM_EXPERTS + idxs[j]], ws[j] / s
                )
                plsc.store_scatter(
                    ivm, [toks * TOP_K + j], idxs[j]
                )
            return _
        lax.fori_loop(0, BT // L, group, 0)

        pltpu.sync_copy(wvm, wout_hbm.at[pl.ds(blk * BT * NUM_EXPERTS,
                                               BT * NUM_EXPERTS)])
        pltpu.sync_copy(ivm, iout_hbm.at[pl.ds(blk * BT * TOP_K,
                                               BT * TOP_K)])
        return 0

    lax.fori_loop(0, blocks_per_w, do_block, 0)


@jax.jit
def kernel(input, W):
    b, s, d = input.shape
    e = W.shape[0]
    t = b * s
    x2 = input.reshape(t, d)
    nblk = t // BT

    logits_blocked = pl.pallas_call(
        _logits_block,
        grid=(t // TCB,),
        in_specs=[
            pl.BlockSpec((TCB, d), lambda i: (i, 0)),
            pl.BlockSpec((e, d), lambda i: (0, 0)),
        ],
        out_specs=pl.BlockSpec((2, 1, e * BT), lambda i: (i, 0, 0)),
        out_shape=jax.ShapeDtypeStruct((nblk, 1, e * BT), jnp.float32),
    )(x2, W)

    mesh = plsc.VectorSubcoreMesh(core_axis_name="c", subcore_axis_name="s")
    weights, idx = pl.kernel(
        _sc_body,
        out_type=[
            jax.ShapeDtypeStruct((t * e,), jnp.float32),
            jax.ShapeDtypeStruct((t * TOP_K,), jnp.int32),
        ],
        mesh=mesh,
        compiler_params=pltpu.CompilerParams(needs_layout_passes=False),
        scratch_types=[
            pltpu.VMEM((e * BT,), jnp.float32),
            pltpu.VMEM((BT * e,), jnp.float32),
            pltpu.VMEM((BT * TOP_K,), jnp.int32),
        ],
    )(logits_blocked)

    return weights.reshape(b, s, e), idx.reshape(b, s, TOP_K)


# speedup vs baseline: 2.4270x; 1.0605x over previous
import jax
import jax.numpy as jnp
from jax import lax
from jax.experimental import pallas as pl
from jax.experimental.pallas import tpu as pltpu
from jax.experimental.pallas import tpu_sc as plsc

NUM_EXPERTS = 64
TOP_K = 8
BT = 512
TCB = 1024
NW = 32
L = 16


def _logits_block(x_ref, w_ref, out_ref):
    x = x_ref[...]
    w = w_ref[...]
    lt = lax.dot_general(
        w, x, (((1,), (1,)), ((), ())), preferred_element_type=jnp.float32
    )
    out_ref[0] = lt[:, :BT].reshape(1, NUM_EXPERTS * BT)
    out_ref[1] = lt[:, BT:].reshape(1, NUM_EXPERTS * BT)


def _sc_body(lin_hbm, wout_hbm, iout_hbm, lvm, wvm, ivm):
    nblk = lin_hbm.shape[0]
    blocks_per_w = nblk // NW
    cid = lax.axis_index("c")
    sid = lax.axis_index("s")
    wid = sid * 2 + cid
    lane = lax.broadcasted_iota(jnp.int32, (L,), 0)
    neg_inf = jnp.full((L,), -jnp.inf, dtype=jnp.float32)
    zeros = jnp.zeros((L,), jnp.float32)

    def do_block(c, _):
        blk = wid * blocks_per_w + c
        pltpu.sync_copy(lin_hbm.at[blk, 0], lvm)

        def zrow(r, _):
            for u in range(8):
                wvm[pl.ds((r * 8 + u) * L, L)] = zeros
            return _
        lax.fori_loop(0, BT * NUM_EXPERTS // (8 * L), zrow, 0)

        def group(g, _):
            base = g * L
            toks = base + lane
            vals = []
            idxs = []
            for j in range(TOP_K):
                ms = []
                ids = []
                for ch in range(4):
                    m = neg_inf
                    idx = jnp.zeros((L,), jnp.int32)
                    for e in range(16):
                        ee = ch * 16 + e
                        v = lvm[pl.ds(ee * BT + base, L)]
                        upd = v > m
                        m = jnp.where(upd, v, m)
                        idx = jnp.where(upd, jnp.int32(ee), idx)
                    ms.append(m)
                    ids.append(idx)
                u = ms[1] > ms[0]
                m01 = jnp.where(u, ms[1], ms[0])
                i01 = jnp.where(u, ids[1], ids[0])
                u = ms[3] > ms[2]
                m23 = jnp.where(u, ms[3], ms[2])
                i23 = jnp.where(u, ids[3], ids[2])
                u = m23 > m01
                m = jnp.where(u, m23, m01)
                idx = jnp.where(u, i23, i01)
                vals.append(m)
                idxs.append(idx)
                if j + 1 < TOP_K:
                    plsc.store_scatter(lvm, [idx * BT + toks], neg_inf)
            ws = [jnp.exp(v - vals[0]) for v in vals]
            s = ws[0]
            for j in range(1, TOP_K):
                s = s + ws[j]
            for j in range(TOP_K):
                plsc.store_scatter(
                    wvm, [toks * NUM_EXPERTS + idxs[j]], ws[j] / s
                )
                plsc.store_scatter(
                    ivm, [toks * TOP_K + j], idxs[j]
                )
            return _
        lax.fori_loop(0, BT // L, group, 0)

        pltpu.sync_copy(wvm, wout_hbm.at[pl.ds(blk * BT * NUM_EXPERTS,
                                               BT * NUM_EXPERTS)])
        pltpu.sync_copy(ivm, iout_hbm.at[pl.ds(blk * BT * TOP_K,
                                               BT * TOP_K)])
        return 0

    lax.fori_loop(0, blocks_per_w, do_block, 0)


@jax.jit
def kernel(input, W):
    b, s, d = input.shape
    e = W.shape[0]
    t = b * s
    x2 = input.reshape(t, d)
    nblk = t // BT

    logits_blocked = pl.pallas_call(
        _logits_block,
        grid=(t // TCB,),
        in_specs=[
            pl.BlockSpec((TCB, d), lambda i: (i, 0)),
            pl.BlockSpec((e, d), lambda i: (0, 0)),
        ],
        out_specs=pl.BlockSpec((2, 1, e * BT), lambda i: (i, 0, 0)),
        out_shape=jax.ShapeDtypeStruct((nblk, 1, e * BT), jnp.float32),
    )(x2, W)

    mesh = plsc.VectorSubcoreMesh(core_axis_name="c", subcore_axis_name="s")
    weights, idx = pl.kernel(
        _sc_body,
        out_type=[
            jax.ShapeDtypeStruct((t * e,), jnp.float32),
            jax.ShapeDtypeStruct((t * TOP_K,), jnp.int32),
        ],
        mesh=mesh,
        compiler_params=pltpu.CompilerParams(needs_layout_passes=False),
        scratch_types=[
            pltpu.VMEM((e * BT,), jnp.float32),
            pltpu.VMEM((BT * e,), jnp.float32),
            pltpu.VMEM((BT * TOP_K,), jnp.int32),
        ],
    )(logits_blocked)

    return weights.reshape(b, s, e), idx.reshape(b, s, TOP_K)


# speedup vs baseline: 2.5251x; 1.0404x over previous
import jax
import jax.numpy as jnp
from jax import lax
from jax.experimental import pallas as pl
from jax.experimental.pallas import tpu as pltpu
from jax.experimental.pallas import tpu_sc as plsc

NUM_EXPERTS = 64
TOP_K = 8
BT = 512
TCB = 1024
NW = 32
L = 16


def _logits_block(x_ref, w_ref, out_ref):
    x = x_ref[...]
    w = w_ref[...]
    lt = lax.dot_general(
        w, x, (((1,), (1,)), ((), ())), preferred_element_type=jnp.float32
    )
    out_ref[0] = lt[:, :BT].reshape(1, NUM_EXPERTS * BT)
    out_ref[1] = lt[:, BT:].reshape(1, NUM_EXPERTS * BT)


def _sc_body(lin_hbm, wout_hbm, iout_hbm, lvm, wvm, ivm):
    nblk = lin_hbm.shape[0]
    blocks_per_w = nblk // NW
    cid = lax.axis_index("c")
    sid = lax.axis_index("s")
    wid = sid * 2 + cid
    lane = lax.broadcasted_iota(jnp.int32, (L,), 0)
    neg_inf = jnp.full((L,), -jnp.inf, dtype=jnp.float32)
    zeros = jnp.zeros((L,), jnp.float32)

    def do_block(c, _):
        blk = wid * blocks_per_w + c
        pltpu.sync_copy(lin_hbm.at[blk, 0], lvm)

        def zrow(r, _):
            for u in range(8):
                wvm[pl.ds((r * 8 + u) * L, L)] = zeros
            return _
        lax.fori_loop(0, BT * NUM_EXPERTS // (8 * L), zrow, 0)

        def group(g, _):
            base = g * L
            toks = base + lane
            def merge(ms, ids):
                u = ms[1] > ms[0]
                m01 = jnp.where(u, ms[1], ms[0])
                i01 = jnp.where(u, ids[1], ids[0])
                u = ms[3] > ms[2]
                m23 = jnp.where(u, ms[3], ms[2])
                i23 = jnp.where(u, ids[3], ids[2])
                u = m23 > m01
                return jnp.where(u, m23, m01), jnp.where(u, i23, i01)

            ms = []
            ids = []
            for ch in range(4):
                m = neg_inf
                idx = jnp.zeros((L,), jnp.int32)
                for e in range(16):
                    ee = ch * 16 + e
                    v = lvm[pl.ds(ee * BT + base, L)]
                    upd = v > m
                    m = jnp.where(upd, v, m)
                    idx = jnp.where(upd, jnp.int32(ee), idx)
                ms.append(m)
                ids.append(idx)
            m, idx = merge(ms, ids)
            vals = [m]
            idxs = [idx]
            for j in range(1, TOP_K):
                prev = idxs[-1]
                plsc.store_scatter(lvm, [prev * BT + toks], neg_inf)
                cstar = lax.shift_right_logical(prev, 4)
                ebase = cstar * jnp.int32(16)
                abase = ebase * jnp.int32(BT) + toks
                rm = neg_inf
                ridx = jnp.zeros((L,), jnp.int32)
                for e in range(16):
                    v = plsc.load_gather(lvm, [abase + jnp.int32(e * BT)])
                    upd = v > rm
                    rm = jnp.where(upd, v, rm)
                    ridx = jnp.where(upd, ebase + jnp.int32(e), ridx)
                nms = []
                nids = []
                for ch in range(4):
                    hit = cstar == ch
                    nms.append(jnp.where(hit, rm, ms[ch]))
                    nids.append(jnp.where(hit, ridx, ids[ch]))
                ms = nms
                ids = nids
                m, idx = merge(ms, ids)
                vals.append(m)
                idxs.append(idx)
            ws = [jnp.exp(v - vals[0]) for v in vals]
            s = ws[0]
            for j in range(1, TOP_K):
                s = s + ws[j]
            for j in range(TOP_K):
                plsc.store_scatter(
                    wvm, [toks * NUM_EXPERTS + idxs[j]], ws[j] / s
                )
                plsc.store_scatter(
                    ivm, [toks * TOP_K + j], idxs[j]
                )
            return _
        lax.fori_loop(0, BT // L, group, 0)

        pltpu.sync_copy(wvm, wout_hbm.at[pl.ds(blk * BT * NUM_EXPERTS,
                                               BT * NUM_EXPERTS)])
        pltpu.sync_copy(ivm, iout_hbm.at[pl.ds(blk * BT * TOP_K,
                                               BT * TOP_K)])
        return 0

    lax.fori_loop(0, blocks_per_w, do_block, 0)


@jax.jit
def kernel(input, W):
    b, s, d = input.shape
    e = W.shape[0]
    t = b * s
    x2 = input.reshape(t, d)
    nblk = t // BT

    logits_blocked = pl.pallas_call(
        _logits_block,
        grid=(t // TCB,),
        in_specs=[
            pl.BlockSpec((TCB, d), lambda i: (i, 0)),
            pl.BlockSpec((e, d), lambda i: (0, 0)),
        ],
        out_specs=pl.BlockSpec((2, 1, e * BT), lambda i: (i, 0, 0)),
        out_shape=jax.ShapeDtypeStruct((nblk, 1, e * BT), jnp.float32),
    )(x2, W)

    mesh = plsc.VectorSubcoreMesh(core_axis_name="c", subcore_axis_name="s")
    weights, idx = pl.kernel(
        _sc_body,
        out_type=[
            jax.ShapeDtypeStruct((t * e,), jnp.float32),
            jax.ShapeDtypeStruct((t * TOP_K,), jnp.int32),
        ],
        mesh=mesh,
        compiler_params=pltpu.CompilerParams(needs_layout_passes=False),
        scratch_types=[
            pltpu.VMEM((e * BT,), jnp.float32),
            pltpu.VMEM((BT * e,), jnp.float32),
            pltpu.VMEM((BT * TOP_K,), jnp.int32),
        ],
    )(logits_blocked)

    return weights.reshape(b, s, e), idx.reshape(b, s, TOP_K)
